# baseline (speedup 1.0000x reference)
import jax
import jax.numpy as jnp
from jax import lax
from jax.experimental import pallas as pl
from jax.experimental.pallas import tpu as pltpu
from jax.experimental.pallas import tpu_sc as plsc

NUM_REL = 6
NUM_HEADS = 32
SEQ = 64
LANES = 16
NW = 32
ROWS = SEQ // NW
CHUNKS = ROWS * SEQ // LANES


def _sc_relation_bias(w, idx):
    mesh = plsc.VectorSubcoreMesh(core_axis_name="c", subcore_axis_name="s")

    def body(w_hbm, idx_hbm, out_hbm, w_v, idx_v, out_v, wt_v, sem_w, sem_i, sem_o):
        wid = lax.axis_index("s") * 2 + lax.axis_index("c")
        r0 = wid * ROWS
        cw = pltpu.async_copy(w_hbm, w_v, sem_w)
        ci = pltpu.async_copy(idx_hbm.at[pl.ds(r0, ROWS)], idx_v, sem_i)
        cw.wait()
        ci.wait()
        rvec = jnp.minimum(lax.iota(jnp.int32, LANES), NUM_REL - 1)
        for h in range(NUM_HEADS):
            hvec = jnp.full((LANES,), h, dtype=jnp.int32)
            wt_v[h] = plsc.load_gather(w_v, [rvec, hvec])
        chunks = [
            idx_v[c // (SEQ // LANES), pl.ds((c % (SEQ // LANES)) * LANES, LANES)]
            for c in range(CHUNKS)
        ]
        HALF = NUM_HEADS // 2

        @plsc.parallel_loop(0, HALF, step=1, unroll=8)
        def h_body_lo(h):
            hvec = jnp.full((LANES,), h, dtype=jnp.int32)
            for c in range(CHUNKS):
                sl = pl.ds((c % (SEQ // LANES)) * LANES, LANES)
                out_v[h, c // (SEQ // LANES), sl] = plsc.load_gather(
                    wt_v, [hvec, chunks[c]]
                )

        co = pltpu.async_copy(
            out_v.at[pl.ds(0, HALF)], out_hbm.at[pl.ds(0, HALF), pl.ds(r0, ROWS), :], sem_o
        )

        @plsc.parallel_loop(HALF, NUM_HEADS, step=1, unroll=8)
        def h_body_hi(h):
            hvec = jnp.full((LANES,), h, dtype=jnp.int32)
            for c in range(CHUNKS):
                sl = pl.ds((c % (SEQ // LANES)) * LANES, LANES)
                out_v[h, c // (SEQ // LANES), sl] = plsc.load_gather(
                    wt_v, [hvec, chunks[c]]
                )

        co.wait()
        pltpu.sync_copy(
            out_v.at[pl.ds(HALF, HALF)], out_hbm.at[pl.ds(HALF, HALF), pl.ds(r0, ROWS), :]
        )

    return pl.kernel(
        body,
        mesh=mesh,
        compiler_params=pltpu.CompilerParams(needs_layout_passes=False),
        out_type=jax.ShapeDtypeStruct((NUM_HEADS, SEQ, SEQ), jnp.float32),
        scratch_types=[
            pltpu.VMEM((NUM_REL, NUM_HEADS), jnp.float32),
            pltpu.VMEM((ROWS, SEQ), jnp.int32),
            pltpu.VMEM((NUM_HEADS, ROWS, SEQ), jnp.float32),
            pltpu.VMEM((NUM_HEADS, LANES), jnp.float32),
            pltpu.SemaphoreType.DMA,
            pltpu.SemaphoreType.DMA,
            pltpu.SemaphoreType.DMA,
        ],
    )(w, idx)


def kernel(embedding_weight, relation_index):
    w = embedding_weight.astype(jnp.float32)
    idx = relation_index.astype(jnp.int32)
    return _sc_relation_bias(w, idx)


# speedup vs baseline: 1.0044x; 1.0044x over previous
import jax
import jax.numpy as jnp
from jax import lax
from jax.experimental import pallas as pl
from jax.experimental.pallas import tpu as pltpu
from jax.experimental.pallas import tpu_sc as plsc

NUM_REL = 6
NUM_HEADS = 32
SEQ = 64
LANES = 16
NW = 32
ROWS = SEQ // NW
CHUNKS = ROWS * SEQ // LANES


def _sc_relation_bias(w, idx):
    mesh = plsc.VectorSubcoreMesh(core_axis_name="c", subcore_axis_name="s")

    def body(w_hbm, idx_hbm, out_hbm, w_v, idx_v, out_v, wt_v, sem_w, sem_i):
        wid = lax.axis_index("s") * 2 + lax.axis_index("c")
        r0 = wid * ROWS
        cw = pltpu.async_copy(w_hbm, w_v, sem_w)
        ci = pltpu.async_copy(idx_hbm.at[pl.ds(r0, ROWS)], idx_v, sem_i)
        cw.wait()
        ci.wait()
        rvec = jnp.minimum(lax.iota(jnp.int32, LANES), NUM_REL - 1)
        for h in range(NUM_HEADS):
            hvec = jnp.full((LANES,), h, dtype=jnp.int32)
            wt_v[h] = plsc.load_gather(w_v, [rvec, hvec])
        chunks = [
            idx_v[c // (SEQ // LANES), pl.ds((c % (SEQ // LANES)) * LANES, LANES)]
            for c in range(CHUNKS)
        ]
        @plsc.parallel_loop(0, NUM_HEADS, step=1, unroll=8)
        def h_body(h):
            hvec = jnp.full((LANES,), h, dtype=jnp.int32)
            for c in range(CHUNKS):
                sl = pl.ds((c % (SEQ // LANES)) * LANES, LANES)
                out_v[h, c // (SEQ // LANES), sl] = plsc.load_gather(
                    wt_v, [hvec, chunks[c]]
                )

        pltpu.sync_copy(out_v, out_hbm.at[:, pl.ds(r0, ROWS), :])

    return pl.kernel(
        body,
        mesh=mesh,
        compiler_params=pltpu.CompilerParams(needs_layout_passes=False),
        out_type=jax.ShapeDtypeStruct((NUM_HEADS, SEQ, SEQ), jnp.float32),
        scratch_types=[
            pltpu.VMEM((NUM_REL, NUM_HEADS), jnp.float32),
            pltpu.VMEM((ROWS, SEQ), jnp.int32),
            pltpu.VMEM((NUM_HEADS, ROWS, SEQ), jnp.float32),
            pltpu.VMEM((NUM_HEADS, LANES), jnp.float32),
            pltpu.SemaphoreType.DMA,
            pltpu.SemaphoreType.DMA,
        ],
    )(w, idx)


def kernel(embedding_weight, relation_index):
    w = embedding_weight.astype(jnp.float32)
    idx = relation_index.astype(jnp.int32)
    return _sc_relation_bias(w, idx)
